# Initial kernel scaffold; baseline (speedup 1.0000x reference)
import jax
import jax.numpy as jnp
from jax.experimental import pallas as pl


def kernel(points_xyz, points_features, skipped_xyz, skipped_features, W1a, b1a, W1b, b1b, gamma, beta):
    raise NotImplementedError("write your pallas kernel here")


# speedup vs baseline: 18.3313x; 18.3313x over previous
import functools

import jax
import jax.numpy as jnp
from jax.experimental import pallas as pl
from jax.experimental.pallas import tpu as pltpu

_EPS_BN = 1e-5
_HI = jax.lax.Precision.HIGHEST


def _stats_body(pf_ref, sf_ref, w1a_ref, b1a_ref, w1b_ref, b1b_ref,
                gamma_ref, beta_ref,
                tmp1_ref, scale1_ref, shift1_ref, scale2_ref, shift2_ref,
                acc_ref, *, nb, nt, n1_rows, n2_rows):
    b = pl.program_id(0)
    t = pl.program_id(1)

    @pl.when(jnp.logical_and(b == 0, t == 0))
    def _init():
        acc_ref[...] = jnp.zeros_like(acc_ref)

    @pl.when(t == 0)
    def _branch1():
        t1 = jnp.dot(pf_ref[0], w1a_ref[...], precision=_HI,
                     preferred_element_type=jnp.float32) + b1a_ref[...]
        tmp1_ref[0] = t1
        acc_ref[0:1, :] += jnp.sum(t1, axis=0, keepdims=True)
        acc_ref[1:2, :] += jnp.sum(t1 * t1, axis=0, keepdims=True)

    t2 = jnp.dot(sf_ref[0], w1b_ref[...], precision=_HI,
                 preferred_element_type=jnp.float32) + b1b_ref[...]
    acc_ref[2:3, :] += jnp.sum(t2, axis=0, keepdims=True)
    acc_ref[3:4, :] += jnp.sum(t2 * t2, axis=0, keepdims=True)

    @pl.when(jnp.logical_and(b == nb - 1, t == nt - 1))
    def _finalize():
        gamma = gamma_ref[...]
        beta = beta_ref[...]
        mean1 = acc_ref[0:1, :] / n2_rows
        var1 = acc_ref[1:2, :] / n2_rows - mean1 * mean1
        s1 = gamma * jax.lax.rsqrt(var1 + _EPS_BN)
        scale1_ref[...] = s1
        shift1_ref[...] = beta - mean1 * s1
        mean2 = acc_ref[2:3, :] / n1_rows
        var2 = acc_ref[3:4, :] / n1_rows - mean2 * mean2
        s2 = gamma * jax.lax.rsqrt(var2 + _EPS_BN)
        scale2_ref[...] = s2
        shift2_ref[...] = beta - mean2 * s2


def _main_body(sxyz_ref, pxt_ref, tmp1_ref, sf_ref, w1b_ref, b1b_ref,
               scale1_ref, shift1_ref, scale2_ref, shift2_ref,
               out_ref, feat2_ref, *, n2, tile):
    t = pl.program_id(1)

    @pl.when(t == 0)
    def _norm_feat2():
        feat2_ref[...] = jnp.maximum(
            tmp1_ref[0] * scale1_ref[...] + shift1_ref[...], 0.0)

    sx = sxyz_ref[0]
    px = pxt_ref[0]
    a2 = jnp.sum(sx * sx, axis=1, keepdims=True)
    b2 = jnp.sum(px * px, axis=0, keepdims=True)
    ab = jnp.dot(sx, px, precision=jax.lax.Precision.DEFAULT,
                 preferred_element_type=jnp.float32)
    sq = jnp.maximum(a2 + b2 - 2.0 * ab, 1e-12)

    iota = jax.lax.broadcasted_iota(jnp.int32, (tile, n2), 1)
    s_acc = jnp.zeros((tile, n2), jnp.float32)
    wsum = jnp.zeros((tile, 1), jnp.float32)
    for j in range(3):
        m = jnp.min(sq, axis=1, keepdims=True)
        am = jnp.min(jnp.where(sq == m, iota, n2), axis=1, keepdims=True)
        sel = iota == am
        w = 1.0 / (jnp.sqrt(m) + 1e-8)
        s_acc = s_acc + jnp.where(sel, w, 0.0)
        wsum = wsum + w
        if j < 2:
            sq = jnp.where(sel, jnp.inf, sq)
    s_mat = s_acc / wsum

    interp = jnp.dot(s_mat, feat2_ref[...], precision=_HI,
                     preferred_element_type=jnp.float32)
    t2 = jnp.dot(sf_ref[0], w1b_ref[...], precision=_HI,
                 preferred_element_type=jnp.float32) + b1b_ref[...]
    out_ref[0] = interp + jnp.maximum(t2 * scale2_ref[...] + shift2_ref[...],
                                      0.0)


@jax.jit
def kernel(points_xyz, points_features, skipped_xyz, skipped_features,
           W1a, b1a, W1b, b1b, gamma, beta):
    B, N2, Cin = points_features.shape
    _, N1, C = skipped_features.shape
    TILE = 1024
    NT = N1 // TILE

    b1a2 = b1a.reshape(1, C)
    b1b2 = b1b.reshape(1, C)
    gamma2 = gamma.reshape(1, C)
    beta2 = beta.reshape(1, C)
    pxt = jnp.transpose(points_xyz, (0, 2, 1))

    vec = pl.BlockSpec((1, C), lambda b, t: (0, 0))
    stats = functools.partial(_stats_body, nb=B, nt=NT,
                              n1_rows=float(B * N1), n2_rows=float(B * N2))
    tmp1, scale1, shift1, scale2, shift2 = pl.pallas_call(
        stats,
        grid=(B, NT),
        in_specs=[
            pl.BlockSpec((1, N2, Cin), lambda b, t: (b, 0, 0)),
            pl.BlockSpec((1, TILE, C), lambda b, t: (b, t, 0)),
            pl.BlockSpec((Cin, C), lambda b, t: (0, 0)),
            vec, pl.BlockSpec((C, C), lambda b, t: (0, 0)), vec, vec, vec,
        ],
        out_specs=[
            pl.BlockSpec((1, N2, C), lambda b, t: (b, 0, 0)),
            vec, vec, vec, vec,
        ],
        out_shape=[
            jax.ShapeDtypeStruct((B, N2, C), jnp.float32),
            jax.ShapeDtypeStruct((1, C), jnp.float32),
            jax.ShapeDtypeStruct((1, C), jnp.float32),
            jax.ShapeDtypeStruct((1, C), jnp.float32),
            jax.ShapeDtypeStruct((1, C), jnp.float32),
        ],
        scratch_shapes=[pltpu.VMEM((4, C), jnp.float32)],
    )(points_features, skipped_features, W1a, b1a2, W1b, b1b2, gamma2, beta2)

    main = functools.partial(_main_body, n2=N2, tile=TILE)
    out = pl.pallas_call(
        main,
        grid=(B, NT),
        in_specs=[
            pl.BlockSpec((1, TILE, 3), lambda b, t: (b, t, 0)),
            pl.BlockSpec((1, 3, N2), lambda b, t: (b, 0, 0)),
            pl.BlockSpec((1, N2, C), lambda b, t: (b, 0, 0)),
            pl.BlockSpec((1, TILE, C), lambda b, t: (b, t, 0)),
            pl.BlockSpec((C, C), lambda b, t: (0, 0)),
            vec, vec, vec, vec, vec,
        ],
        out_specs=pl.BlockSpec((1, TILE, C), lambda b, t: (b, t, 0)),
        out_shape=jax.ShapeDtypeStruct((B, N1, C), jnp.float32),
        scratch_shapes=[pltpu.VMEM((N2, C), jnp.float32)],
    )(skipped_xyz, pxt, tmp1, skipped_features, W1b, b1b2,
      scale1, shift1, scale2, shift2)

    return (skipped_xyz, out)


# speedup vs baseline: 28.7436x; 1.5680x over previous
import functools

import jax
import jax.numpy as jnp
from jax.experimental import pallas as pl
from jax.experimental.pallas import tpu as pltpu

_EPS_BN = 1e-5
_HI = jax.lax.Precision.DEFAULT


def _stats_body(pf_ref, sf_ref, w1a_ref, b1a_ref, w1b_ref, b1b_ref,
                gamma_ref, beta_ref,
                tmp1_ref, scale1_ref, shift1_ref, scale2_ref, shift2_ref,
                acc_ref, *, nb, nt, n1_rows, n2_rows):
    b = pl.program_id(0)
    t = pl.program_id(1)

    @pl.when(jnp.logical_and(b == 0, t == 0))
    def _init():
        acc_ref[...] = jnp.zeros_like(acc_ref)

    @pl.when(t == 0)
    def _branch1():
        t1 = jnp.dot(pf_ref[0], w1a_ref[...], precision=_HI,
                     preferred_element_type=jnp.float32) + b1a_ref[...]
        tmp1_ref[0] = t1
        acc_ref[0:1, :] += jnp.sum(t1, axis=0, keepdims=True)
        acc_ref[1:2, :] += jnp.sum(t1 * t1, axis=0, keepdims=True)

    t2 = jnp.dot(sf_ref[0], w1b_ref[...], precision=_HI,
                 preferred_element_type=jnp.float32) + b1b_ref[...]
    acc_ref[2:3, :] += jnp.sum(t2, axis=0, keepdims=True)
    acc_ref[3:4, :] += jnp.sum(t2 * t2, axis=0, keepdims=True)

    @pl.when(jnp.logical_and(b == nb - 1, t == nt - 1))
    def _finalize():
        gamma = gamma_ref[...]
        beta = beta_ref[...]
        mean1 = acc_ref[0:1, :] / n2_rows
        var1 = acc_ref[1:2, :] / n2_rows - mean1 * mean1
        s1 = gamma * jax.lax.rsqrt(var1 + _EPS_BN)
        scale1_ref[...] = s1
        shift1_ref[...] = beta - mean1 * s1
        mean2 = acc_ref[2:3, :] / n1_rows
        var2 = acc_ref[3:4, :] / n1_rows - mean2 * mean2
        s2 = gamma * jax.lax.rsqrt(var2 + _EPS_BN)
        scale2_ref[...] = s2
        shift2_ref[...] = beta - mean2 * s2


def _main_body(sxyz_ref, pxt_ref, tmp1_ref, sf_ref, w1b_ref, b1b_ref,
               scale1_ref, shift1_ref, scale2_ref, shift2_ref,
               out_ref, feat2_ref, *, n2, tile):
    t = pl.program_id(1)

    @pl.when(t == 0)
    def _norm_feat2():
        feat2_ref[...] = jnp.maximum(
            tmp1_ref[0] * scale1_ref[...] + shift1_ref[...], 0.0)

    sx = sxyz_ref[0]
    px = pxt_ref[0]
    a2 = jnp.sum(sx * sx, axis=1, keepdims=True)
    b2 = jnp.sum(px * px, axis=0, keepdims=True)
    ab = jnp.dot(sx, px, precision=jax.lax.Precision.DEFAULT,
                 preferred_element_type=jnp.float32)
    sq = jnp.maximum(a2 + b2 - 2.0 * ab, 1e-12)

    iota = jax.lax.broadcasted_iota(jnp.int32, (tile, n2), 1)
    s_acc = jnp.zeros((tile, n2), jnp.float32)
    wsum = jnp.zeros((tile, 1), jnp.float32)
    for j in range(3):
        m = jnp.min(sq, axis=1, keepdims=True)
        am = jnp.min(jnp.where(sq == m, iota, n2), axis=1, keepdims=True)
        sel = iota == am
        w = 1.0 / (jnp.sqrt(m) + 1e-8)
        s_acc = s_acc + jnp.where(sel, w, 0.0)
        wsum = wsum + w
        if j < 2:
            sq = jnp.where(sel, jnp.inf, sq)
    s_mat = s_acc / wsum

    interp = jnp.dot(s_mat, feat2_ref[...], precision=_HI,
                     preferred_element_type=jnp.float32)
    t2 = jnp.dot(sf_ref[0], w1b_ref[...], precision=_HI,
                 preferred_element_type=jnp.float32) + b1b_ref[...]
    out_ref[0] = interp + jnp.maximum(t2 * scale2_ref[...] + shift2_ref[...],
                                      0.0)


@jax.jit
def kernel(points_xyz, points_features, skipped_xyz, skipped_features,
           W1a, b1a, W1b, b1b, gamma, beta):
    B, N2, Cin = points_features.shape
    _, N1, C = skipped_features.shape
    TILE = 1024
    NT = N1 // TILE

    b1a2 = b1a.reshape(1, C)
    b1b2 = b1b.reshape(1, C)
    gamma2 = gamma.reshape(1, C)
    beta2 = beta.reshape(1, C)
    pxt = jnp.transpose(points_xyz, (0, 2, 1))

    vec = pl.BlockSpec((1, C), lambda b, t: (0, 0))
    stats = functools.partial(_stats_body, nb=B, nt=NT,
                              n1_rows=float(B * N1), n2_rows=float(B * N2))
    tmp1, scale1, shift1, scale2, shift2 = pl.pallas_call(
        stats,
        grid=(B, NT),
        in_specs=[
            pl.BlockSpec((1, N2, Cin), lambda b, t: (b, 0, 0)),
            pl.BlockSpec((1, TILE, C), lambda b, t: (b, t, 0)),
            pl.BlockSpec((Cin, C), lambda b, t: (0, 0)),
            vec, pl.BlockSpec((C, C), lambda b, t: (0, 0)), vec, vec, vec,
        ],
        out_specs=[
            pl.BlockSpec((1, N2, C), lambda b, t: (b, 0, 0)),
            vec, vec, vec, vec,
        ],
        out_shape=[
            jax.ShapeDtypeStruct((B, N2, C), jnp.float32),
            jax.ShapeDtypeStruct((1, C), jnp.float32),
            jax.ShapeDtypeStruct((1, C), jnp.float32),
            jax.ShapeDtypeStruct((1, C), jnp.float32),
            jax.ShapeDtypeStruct((1, C), jnp.float32),
        ],
        scratch_shapes=[pltpu.VMEM((4, C), jnp.float32)],
    )(points_features, skipped_features, W1a, b1a2, W1b, b1b2, gamma2, beta2)

    main = functools.partial(_main_body, n2=N2, tile=TILE)
    out = pl.pallas_call(
        main,
        grid=(B, NT),
        in_specs=[
            pl.BlockSpec((1, TILE, 3), lambda b, t: (b, t, 0)),
            pl.BlockSpec((1, 3, N2), lambda b, t: (b, 0, 0)),
            pl.BlockSpec((1, N2, C), lambda b, t: (b, 0, 0)),
            pl.BlockSpec((1, TILE, C), lambda b, t: (b, t, 0)),
            pl.BlockSpec((C, C), lambda b, t: (0, 0)),
            vec, vec, vec, vec, vec,
        ],
        out_specs=pl.BlockSpec((1, TILE, C), lambda b, t: (b, t, 0)),
        out_shape=jax.ShapeDtypeStruct((B, N1, C), jnp.float32),
        scratch_shapes=[pltpu.VMEM((N2, C), jnp.float32)],
    )(skipped_xyz, pxt, tmp1, skipped_features, W1b, b1b2,
      scale1, shift1, scale2, shift2)

    return (skipped_xyz, out)


# speedup vs baseline: 33.8369x; 1.1772x over previous
import functools

import jax
import jax.numpy as jnp
from jax.experimental import pallas as pl
from jax.experimental.pallas import tpu as pltpu

_EPS_BN = 1e-5
_HI = jax.lax.Precision.DEFAULT


def _stats_body(pf_ref, sf_ref, w1a_ref, b1a_ref, w1b_ref, b1b_ref,
                gamma_ref, beta_ref,
                tmp1_ref, scale1_ref, shift1_ref, scale2_ref, shift2_ref,
                acc_ref, *, nb, nt, n1_rows, n2_rows):
    b = pl.program_id(0)
    t = pl.program_id(1)

    @pl.when(jnp.logical_and(b == 0, t == 0))
    def _init():
        acc_ref[...] = jnp.zeros_like(acc_ref)

    @pl.when(t == 0)
    def _branch1():
        t1 = jnp.dot(pf_ref[0], w1a_ref[...], precision=_HI,
                     preferred_element_type=jnp.float32) + b1a_ref[...]
        tmp1_ref[0] = t1
        acc_ref[0:1, :] += jnp.sum(t1, axis=0, keepdims=True)
        acc_ref[1:2, :] += jnp.sum(t1 * t1, axis=0, keepdims=True)

    t2 = jnp.dot(sf_ref[0], w1b_ref[...], precision=_HI,
                 preferred_element_type=jnp.float32) + b1b_ref[...]
    acc_ref[2:3, :] += jnp.sum(t2, axis=0, keepdims=True)
    acc_ref[3:4, :] += jnp.sum(t2 * t2, axis=0, keepdims=True)

    @pl.when(jnp.logical_and(b == nb - 1, t == nt - 1))
    def _finalize():
        gamma = gamma_ref[...]
        beta = beta_ref[...]
        mean1 = acc_ref[0:1, :] / n2_rows
        var1 = acc_ref[1:2, :] / n2_rows - mean1 * mean1
        s1 = gamma * jax.lax.rsqrt(var1 + _EPS_BN)
        scale1_ref[...] = s1
        shift1_ref[...] = beta - mean1 * s1
        mean2 = acc_ref[2:3, :] / n1_rows
        var2 = acc_ref[3:4, :] / n1_rows - mean2 * mean2
        s2 = gamma * jax.lax.rsqrt(var2 + _EPS_BN)
        scale2_ref[...] = s2
        shift2_ref[...] = beta - mean2 * s2


def _main_body(sxyz_ref, pxt_ref, tmp1_ref, sf_ref, w1b_ref, b1b_ref,
               scale1_ref, shift1_ref, scale2_ref, shift2_ref,
               out_ref, feat2_ref, *, n2, tile):
    t = pl.program_id(1)

    @pl.when(t == 0)
    def _norm_feat2():
        feat2_ref[...] = jnp.maximum(
            tmp1_ref[0] * scale1_ref[...] + shift1_ref[...], 0.0)

    sx = sxyz_ref[0]
    px = pxt_ref[0]
    a2 = jnp.sum(sx * sx, axis=1, keepdims=True)
    b2 = jnp.sum(px * px, axis=0, keepdims=True)
    ab = jnp.dot(sx, px, precision=jax.lax.Precision.DEFAULT,
                 preferred_element_type=jnp.float32)
    sq = jnp.maximum(a2 + b2 - 2.0 * ab, 1e-12)

    iota = jax.lax.broadcasted_iota(jnp.int32, sq.shape, 1).astype(jnp.float32)
    m1 = jnp.min(sq, axis=1, keepdims=True)
    i1 = jnp.min(jnp.where(sq == m1, iota, jnp.inf), axis=1, keepdims=True)
    sq1 = jnp.where(iota == i1, jnp.inf, sq)
    m2 = jnp.min(sq1, axis=1, keepdims=True)
    i2 = jnp.min(jnp.where(sq1 == m2, iota, jnp.inf), axis=1, keepdims=True)
    sq2 = jnp.where(iota == i2, jnp.inf, sq1)
    m3 = jnp.min(sq2, axis=1, keepdims=True)
    i3 = jnp.min(jnp.where(sq2 == m3, iota, jnp.inf), axis=1, keepdims=True)
    p1 = jnp.sqrt(m1) + 1e-8
    p2 = jnp.sqrt(m2) + 1e-8
    p3 = jnp.sqrt(m3) + 1e-8
    q12 = p1 * p2
    q23 = p2 * p3
    q13 = p1 * p3
    denom = 1.0 / (q23 + q13 + q12)
    wn1 = q23 * denom
    wn2 = q13 * denom
    wn3 = q12 * denom
    s_mat = jnp.where(iota == i1, wn1,
                      jnp.where(iota == i2, wn2,
                                jnp.where(iota == i3, wn3, 0.0)))

    interp = jnp.dot(s_mat, feat2_ref[...], precision=_HI,
                     preferred_element_type=jnp.float32)
    t2 = jnp.dot(sf_ref[0], w1b_ref[...], precision=_HI,
                 preferred_element_type=jnp.float32) + b1b_ref[...]
    out_ref[0] = interp + jnp.maximum(t2 * scale2_ref[...] + shift2_ref[...],
                                      0.0)


@jax.jit
def kernel(points_xyz, points_features, skipped_xyz, skipped_features,
           W1a, b1a, W1b, b1b, gamma, beta):
    B, N2, Cin = points_features.shape
    _, N1, C = skipped_features.shape
    TILE = 1024
    NT = N1 // TILE

    b1a2 = b1a.reshape(1, C)
    b1b2 = b1b.reshape(1, C)
    gamma2 = gamma.reshape(1, C)
    beta2 = beta.reshape(1, C)
    pxt = jnp.transpose(points_xyz, (0, 2, 1))

    vec = pl.BlockSpec((1, C), lambda b, t: (0, 0))
    stats = functools.partial(_stats_body, nb=B, nt=NT,
                              n1_rows=float(B * N1), n2_rows=float(B * N2))
    tmp1, scale1, shift1, scale2, shift2 = pl.pallas_call(
        stats,
        grid=(B, NT),
        in_specs=[
            pl.BlockSpec((1, N2, Cin), lambda b, t: (b, 0, 0)),
            pl.BlockSpec((1, TILE, C), lambda b, t: (b, t, 0)),
            pl.BlockSpec((Cin, C), lambda b, t: (0, 0)),
            vec, pl.BlockSpec((C, C), lambda b, t: (0, 0)), vec, vec, vec,
        ],
        out_specs=[
            pl.BlockSpec((1, N2, C), lambda b, t: (b, 0, 0)),
            vec, vec, vec, vec,
        ],
        out_shape=[
            jax.ShapeDtypeStruct((B, N2, C), jnp.float32),
            jax.ShapeDtypeStruct((1, C), jnp.float32),
            jax.ShapeDtypeStruct((1, C), jnp.float32),
            jax.ShapeDtypeStruct((1, C), jnp.float32),
            jax.ShapeDtypeStruct((1, C), jnp.float32),
        ],
        scratch_shapes=[pltpu.VMEM((4, C), jnp.float32)],
    )(points_features, skipped_features, W1a, b1a2, W1b, b1b2, gamma2, beta2)

    main = functools.partial(_main_body, n2=N2, tile=TILE)
    out = pl.pallas_call(
        main,
        grid=(B, NT),
        in_specs=[
            pl.BlockSpec((1, TILE, 3), lambda b, t: (b, t, 0)),
            pl.BlockSpec((1, 3, N2), lambda b, t: (b, 0, 0)),
            pl.BlockSpec((1, N2, C), lambda b, t: (b, 0, 0)),
            pl.BlockSpec((1, TILE, C), lambda b, t: (b, t, 0)),
            pl.BlockSpec((C, C), lambda b, t: (0, 0)),
            vec, vec, vec, vec, vec,
        ],
        out_specs=pl.BlockSpec((1, TILE, C), lambda b, t: (b, t, 0)),
        out_shape=jax.ShapeDtypeStruct((B, N1, C), jnp.float32),
        scratch_shapes=[pltpu.VMEM((N2, C), jnp.float32)],
    )(skipped_xyz, pxt, tmp1, skipped_features, W1b, b1b2,
      scale1, shift1, scale2, shift2)

    return (skipped_xyz, out)
